# baseline (speedup 1.0000x reference)
import functools

import jax
import jax.numpy as jnp
from jax import lax
from jax.experimental import pallas as pl
from jax.experimental.pallas import tpu as pltpu
from jax.experimental.pallas import tpu_sc as plsc

B, N, K = 4, 2500, 16
EPS = 1e-5
BN = B * N
NODE_TILE = 256
N_PAD = 10240
E = BN * K
E_PAD = N_PAD * K
NCOL = 2560
ROW_TILE = 512

NUM_WORKERS = 32
CHUNK = 128
NCHUNKS = E_PAD // CHUNK
CH_C0 = 56
CH_C1 = (NCHUNKS // 16) - CH_C0


def _dot(a, b):
    return jnp.dot(a, b, preferred_element_type=jnp.float32)


def _mlp_body(x_ref, w1t_ref, b1_ref, g1_ref, be1_ref,
              w2t_ref, b2_ref, g2_ref, be2_ref,
              du1_ref, dv1_ref, ba1_ref,
              h_ref, u1_ref, v1_ref):
    x = x_ref[...]
    h1 = _dot(x, w1t_ref[...]) + b1_ref[...]
    mu = jnp.mean(h1, axis=0, keepdims=True)
    var = jnp.mean(h1 * h1, axis=0, keepdims=True) - mu * mu
    a1 = jnp.maximum(g1_ref[...] * (h1 - mu) * jax.lax.rsqrt(var + EPS)
                     + be1_ref[...], 0.0)
    h2 = _dot(a1, w2t_ref[...]) + b2_ref[...]
    mu2 = jnp.mean(h2, axis=0, keepdims=True)
    var2 = jnp.mean(h2 * h2, axis=0, keepdims=True) - mu2 * mu2
    h = jnp.maximum(g2_ref[...] * (h2 - mu2) * jax.lax.rsqrt(var2 + EPS)
                    + be2_ref[...], 0.0)
    h_ref[...] = h
    u1_ref[...] = _dot(h, du1_ref[...]) + ba1_ref[...]
    v1_ref[...] = _dot(h, dv1_ref[...])


def _run_mlp(x2d, W1, b1, g1, be1, W2, b2, g2, be2, We1a, ba1):
    A1 = We1a[:, :128]
    A2 = We1a[:, 128:]
    du1 = (A1 - A2).T
    dv1 = A2.T
    xp = jnp.pad(x2d, ((0, 0), (0, 6)))
    return pl.pallas_call(
        _mlp_body,
        out_shape=[
            jax.ShapeDtypeStruct((BN, 128), jnp.float32),
            jax.ShapeDtypeStruct((BN, 128), jnp.float32),
            jax.ShapeDtypeStruct((BN, 128), jnp.float32),
        ],
    )(xp, jnp.pad(W1.T, ((0, 6), (0, 0))), b1[None], g1[None], be1[None],
      W2.T, b2[None], g2[None], be2[None], du1, dv1, ba1[None])


def _knn_body(pos_r_ref, pos_c_ref, out_ref):
    b = pl.program_id(0)
    rt = pl.program_id(1)
    pr = pos_r_ref[0]
    pc = pos_c_ref[0]
    sq_r = jnp.sum(pr * pr, axis=0)[:, None]
    sq_c = jnp.sum(pc * pc, axis=0)[None, :]
    d2 = sq_r + sq_c - 2.0 * lax.dot_general(
        pr, pc, (((0,), (0,)), ((), ())),
        preferred_element_type=jnp.float32)
    col = lax.broadcasted_iota(jnp.int32, (ROW_TILE, NCOL), 1)
    row = lax.broadcasted_iota(jnp.int32, (ROW_TILE, NCOL), 0) + rt * ROW_TILE
    bad = (col >= N) | (col == row)
    s = jnp.where(bad, 1e30, d2)
    base = b * N
    for k in range(K):
        rmin = jnp.min(s, axis=1, keepdims=True)
        cand = jnp.where(s == rmin, col, jnp.int32(2 ** 30))
        idx = jnp.min(cand, axis=1, keepdims=True)
        out_ref[0, k, :] = idx[:, 0] + base
        s = jnp.where(col == idx, 1e30, s)


def _run_knn(posT):
    grid = (B, NCOL // ROW_TILE)
    return pl.pallas_call(
        _knn_body,
        grid=grid,
        in_specs=[
            pl.BlockSpec((1, 4, ROW_TILE), lambda b, rt: (b, 0, rt)),
            pl.BlockSpec((1, 4, NCOL), lambda b, rt: (b, 0, 0)),
        ],
        out_specs=pl.BlockSpec((1, K, ROW_TILE), lambda b, rt: (b, 0, rt)),
        out_shape=jax.ShapeDtypeStruct((B, K, NCOL), jnp.int32),
    )(posT, posT)


def _make_gather(C):
    info = plsc.get_sparse_core_info()
    nc, ns = info.num_cores, info.num_subcores
    mesh = plsc.VectorSubcoreMesh(core_axis_name="c", subcore_axis_name="s")

    @functools.partial(
        pl.kernel,
        out_type=jax.ShapeDtypeStruct((E_PAD, C), jnp.float32),
        mesh=mesh,
        scratch_types=[
            pltpu.VMEM((2, CHUNK), jnp.int32),
            pltpu.VMEM((CHUNK, C), jnp.float32),
            pltpu.VMEM((CHUNK, C), jnp.float32),
            pltpu.SemaphoreType.DMA,
            pltpu.SemaphoreType.DMA,
            pltpu.SemaphoreType.DMA,
            pltpu.SemaphoreType.DMA,
        ],
    )
    def gather(idx_hbm, table_hbm, out_hbm, idx_v, rows0, rows1, g0, g1, w0, w1):
        sid = lax.axis_index("s")
        cid = lax.axis_index("c")

        def out_at(c):
            return out_hbm.at[pl.ds(c * CHUNK, CHUNK)]

        def wait_g(slot, rows, sem):
            pltpu.make_async_copy(table_hbm.at[idx_v.at[slot]], rows, sem).wait()

        def wait_w(rows, sem):
            pltpu.make_async_copy(rows, out_at(0), sem).wait()

        def run(first, nchunks):
            pltpu.sync_copy(idx_hbm.at[first], idx_v.at[0])
            pltpu.async_copy(table_hbm.at[idx_v.at[0]], rows0, g0)

            def body(i, _):
                a = first + 2 * i

                @pl.when(i > 0)
                def _():
                    wait_w(rows1, w1)
                pltpu.sync_copy(idx_hbm.at[a + 1], idx_v.at[1])
                pltpu.async_copy(table_hbm.at[idx_v.at[1]], rows1, g1)

                wait_g(0, rows0, g0)
                pltpu.async_copy(rows0, out_at(a), w0)

                wait_w(rows0, w0)
                pltpu.sync_copy(idx_hbm.at[a + 2], idx_v.at[0])
                pltpu.async_copy(table_hbm.at[idx_v.at[0]], rows0, g0)

                wait_g(1, rows1, g1)
                pltpu.async_copy(rows1, out_at(a + 1), w1)
                return _

            lax.fori_loop(0, (nchunks - 2) // 2, body, None)
            last = first + nchunks - 1
            wait_w(rows1, w1)
            pltpu.sync_copy(idx_hbm.at[last], idx_v.at[1])
            pltpu.async_copy(table_hbm.at[idx_v.at[1]], rows1, g1)
            wait_g(0, rows0, g0)
            pltpu.async_copy(rows0, out_at(last - 1), w0)
            wait_g(1, rows1, g1)
            pltpu.async_copy(rows1, out_at(last), w1)
            wait_w(rows0, w0)
            wait_w(rows1, w1)

        @pl.when(cid == 0)
        def _():
            run(sid * CH_C0, CH_C0)

        @pl.when(cid == 1)
        def _():
            run(16 * CH_C0 + sid * CH_C1, CH_C1)

    return gather


def _conv_body(u_ref, g_ref, wbt_ref, bb_ref, dun_ref, ban_ref, dvn_ref,
               u2_ref, v2_ref, C, CO):
    u = u_ref[...]
    g = g_ref[...]
    urep = jnp.broadcast_to(u[:, None, :], (NODE_TILE, K, C))
    urep = urep.reshape(NODE_TILE * K, C)
    m = jnp.maximum(urep + g, 0.0)
    y = jnp.maximum(_dot(m, wbt_ref[...]) + bb_ref[...], 0.0)
    xn = jnp.max(y.reshape(NODE_TILE, K, C), axis=1)
    u2_ref[...] = _dot(xn, dun_ref[...]) + ban_ref[...]
    v2_ref[...] = _dot(xn, dvn_ref[...])


def _run_conv1(u1, g1rows, We1b, bb1, We2a, ba2):
    A1 = We2a[:, :128]
    A2 = We2a[:, 128:]
    body = functools.partial(_conv_body, C=128, CO=256)
    return pl.pallas_call(
        body,
        grid=(N_PAD // NODE_TILE,),
        in_specs=[
            pl.BlockSpec((NODE_TILE, 128), lambda i: (i, 0)),
            pl.BlockSpec((NODE_TILE * K, 128), lambda i: (i, 0)),
            pl.BlockSpec((128, 128), lambda i: (0, 0)),
            pl.BlockSpec((1, 128), lambda i: (0, 0)),
            pl.BlockSpec((128, 256), lambda i: (0, 0)),
            pl.BlockSpec((1, 256), lambda i: (0, 0)),
            pl.BlockSpec((128, 256), lambda i: (0, 0)),
        ],
        out_specs=[
            pl.BlockSpec((NODE_TILE, 256), lambda i: (i, 0)),
            pl.BlockSpec((NODE_TILE, 256), lambda i: (i, 0)),
        ],
        out_shape=[
            jax.ShapeDtypeStruct((N_PAD, 256), jnp.float32),
            jax.ShapeDtypeStruct((N_PAD, 256), jnp.float32),
        ],
    )(u1, g1rows, We1b.T, bb1[None], (A1 - A2).T, ba2[None], A2.T)


def _conv2_body(u_ref, g_ref, wbt_ref, bb_ref, x2_ref):
    C = 256
    u = u_ref[...]
    g = g_ref[...]
    urep = jnp.broadcast_to(u[:, None, :], (NODE_TILE, K, C))
    urep = urep.reshape(NODE_TILE * K, C)
    m = jnp.maximum(urep + g, 0.0)
    y = jnp.maximum(_dot(m, wbt_ref[...]) + bb_ref[...], 0.0)
    x2_ref[...] = jnp.max(y.reshape(NODE_TILE, K, C), axis=1)


def _run_conv2(u2, g2rows, We2b, bb2):
    return pl.pallas_call(
        _conv2_body,
        grid=(N_PAD // NODE_TILE,),
        in_specs=[
            pl.BlockSpec((NODE_TILE, 256), lambda i: (i, 0)),
            pl.BlockSpec((NODE_TILE * K, 256), lambda i: (i, 0)),
            pl.BlockSpec((256, 256), lambda i: (0, 0)),
            pl.BlockSpec((1, 256), lambda i: (0, 0)),
        ],
        out_specs=pl.BlockSpec((NODE_TILE, 256), lambda i: (i, 0)),
        out_shape=jax.ShapeDtypeStruct((N_PAD, 256), jnp.float32),
    )(u2, g2rows, We2b.T, bb2[None])


def _head_body(x_ref, wf1t_ref, bf1_ref, wf2t_ref, bf2_ref, mu_ref, lv_ref):
    x = x_ref[...]
    mean = jnp.mean(x, axis=1)
    mx = jnp.max(x, axis=1)
    xg = jnp.concatenate([mean, mx], axis=1)
    z = jnp.maximum(_dot(xg, wf1t_ref[...]) + bf1_ref[...], 0.0)
    lat = _dot(z, wf2t_ref[...]) + bf2_ref[...]
    mu_ref[...] = lat[:, :256]
    lv_ref[...] = lat[:, 256:]


def _run_head(x2, Wf1, bf1, Wf2, bf2):
    return pl.pallas_call(
        _head_body,
        out_shape=[
            jax.ShapeDtypeStruct((B, 256), jnp.float32),
            jax.ShapeDtypeStruct((B, 256), jnp.float32),
        ],
    )(x2, Wf1.T, bf1[None], Wf2.T, bf2[None])


def kernel(x, W1, b1, g1, be1, W2, b2, g2, be2,
           We1a, ba1, We1b, bb1, We2a, ba2, We2b, bb2,
           Wf1, bf1, Wf2, bf2):
    x2d = x.reshape(BN, 10)
    h, u1, v1 = _run_mlp(x2d, W1, b1, g1, be1, W2, b2, g2, be2, We1a, ba1)

    pos = h[:, :3].reshape(B, N, 3).transpose(0, 2, 1)
    posT = jnp.pad(pos, ((0, 0), (0, 1), (0, NCOL - N)))

    idx_kn = _run_knn(posT)
    src = idx_kn.transpose(0, 2, 1)[:, :N, :].reshape(E)
    src = jnp.pad(src, (0, E_PAD - E))
    src = src.reshape(NCHUNKS, CHUNK)

    gather128 = _make_gather(128)
    gather256 = _make_gather(256)

    g1rows = gather128(src, v1)
    u1p = jnp.pad(u1, ((0, N_PAD - BN), (0, 0)))
    u2, v2 = _run_conv1(u1p, g1rows, We1b, bb1, We2a, ba2)

    g2rows = gather256(src, v2[:BN])
    x2 = _run_conv2(u2, g2rows, We2b, bb2)

    x2b = x2[:BN].reshape(B, N, 256)
    mu_, lv = _run_head(x2b, Wf1, bf1, Wf2, bf2)
    return (mu_, lv)


# speedup vs baseline: 1.2554x; 1.2554x over previous
import functools

import jax
import jax.numpy as jnp
from jax import lax
from jax.experimental import pallas as pl
from jax.experimental.pallas import tpu as pltpu
from jax.experimental.pallas import tpu_sc as plsc

B, N, K = 4, 2500, 16
EPS = 1e-5
BN = B * N
NODE_TILE = 256
N_PAD = 2560
E = BN * K
E_PAD = N_PAD * K
NCOL = 2560
ROW_TILE = 512

NUM_WORKERS = 32
CHUNK = 128
NCHUNKS = E_PAD // CHUNK
CH_C0 = 10
CH_C1 = (NCHUNKS // 16) - CH_C0


def _dot(a, b):
    return jnp.dot(a, b, preferred_element_type=jnp.float32)


def _mlp_body(x_ref, w1t_ref, b1_ref, g1_ref, be1_ref,
              w2t_ref, b2_ref, g2_ref, be2_ref,
              du1_ref, dv1_ref, ba1_ref,
              h_ref, u1_ref, v1_ref):
    x = x_ref[...]
    h1 = _dot(x, w1t_ref[...]) + b1_ref[...]
    mu = jnp.mean(h1, axis=0, keepdims=True)
    var = jnp.mean(h1 * h1, axis=0, keepdims=True) - mu * mu
    a1 = jnp.maximum(g1_ref[...] * (h1 - mu) * jax.lax.rsqrt(var + EPS)
                     + be1_ref[...], 0.0)
    h2 = _dot(a1, w2t_ref[...]) + b2_ref[...]
    mu2 = jnp.mean(h2, axis=0, keepdims=True)
    var2 = jnp.mean(h2 * h2, axis=0, keepdims=True) - mu2 * mu2
    h = jnp.maximum(g2_ref[...] * (h2 - mu2) * jax.lax.rsqrt(var2 + EPS)
                    + be2_ref[...], 0.0)
    h_ref[...] = h
    u1_ref[...] = _dot(h, du1_ref[...]) + ba1_ref[...]
    v1_ref[...] = _dot(h, dv1_ref[...])


def _run_mlp(x2d, W1, b1, g1, be1, W2, b2, g2, be2, We1a, ba1):
    A1 = We1a[:, :128]
    A2 = We1a[:, 128:]
    du1 = (A1 - A2).T
    dv1 = A2.T
    xp = jnp.pad(x2d, ((0, 0), (0, 6)))
    return pl.pallas_call(
        _mlp_body,
        out_shape=[
            jax.ShapeDtypeStruct((BN, 128), jnp.float32),
            jax.ShapeDtypeStruct((BN, 128), jnp.float32),
            jax.ShapeDtypeStruct((BN, 128), jnp.float32),
        ],
    )(xp, jnp.pad(W1.T, ((0, 6), (0, 0))), b1[None], g1[None], be1[None],
      W2.T, b2[None], g2[None], be2[None], du1, dv1, ba1[None])


def _knn_body(pos_r_ref, pos_c_ref, out_ref):
    b = pl.program_id(0)
    rt = pl.program_id(1)
    pr = pos_r_ref[0]
    pc = pos_c_ref[0]
    sq_r = jnp.sum(pr * pr, axis=0)[:, None]
    sq_c = jnp.sum(pc * pc, axis=0)[None, :]
    d2 = sq_r + sq_c - 2.0 * lax.dot_general(
        pr, pc, (((0,), (0,)), ((), ())),
        preferred_element_type=jnp.float32)
    col = lax.broadcasted_iota(jnp.int32, (ROW_TILE, NCOL), 1)
    row = lax.broadcasted_iota(jnp.int32, (ROW_TILE, NCOL), 0) + rt * ROW_TILE
    bad = (col >= N) | (col == row)
    s = jnp.where(bad, 1e30, d2)
    base = b * N
    for k in range(K):
        rmin = jnp.min(s, axis=1, keepdims=True)
        cand = jnp.where(s == rmin, col, jnp.int32(2 ** 30))
        idx = jnp.min(cand, axis=1, keepdims=True)
        out_ref[0, k, :] = idx[:, 0] + base
        s = jnp.where(col == idx, 1e30, s)


def _run_knn(posT):
    grid = (1, NCOL // ROW_TILE)
    return pl.pallas_call(
        _knn_body,
        grid=grid,
        in_specs=[
            pl.BlockSpec((1, 4, ROW_TILE), lambda b, rt: (b, 0, rt)),
            pl.BlockSpec((1, 4, NCOL), lambda b, rt: (b, 0, 0)),
        ],
        out_specs=pl.BlockSpec((1, K, ROW_TILE), lambda b, rt: (b, 0, rt)),
        out_shape=jax.ShapeDtypeStruct((1, K, NCOL), jnp.int32),
    )(posT, posT)


def _make_gather(C):
    info = plsc.get_sparse_core_info()
    nc, ns = info.num_cores, info.num_subcores
    mesh = plsc.VectorSubcoreMesh(core_axis_name="c", subcore_axis_name="s")

    @functools.partial(
        pl.kernel,
        out_type=jax.ShapeDtypeStruct((E_PAD, C), jnp.float32),
        mesh=mesh,
        scratch_types=[
            pltpu.VMEM((2, CHUNK), jnp.int32),
            pltpu.VMEM((CHUNK, C), jnp.float32),
            pltpu.VMEM((CHUNK, C), jnp.float32),
            pltpu.SemaphoreType.DMA,
            pltpu.SemaphoreType.DMA,
            pltpu.SemaphoreType.DMA,
            pltpu.SemaphoreType.DMA,
        ],
    )
    def gather(idx_hbm, table_hbm, out_hbm, idx_v, rows0, rows1, g0, g1, w0, w1):
        sid = lax.axis_index("s")
        cid = lax.axis_index("c")

        def out_at(c):
            return out_hbm.at[pl.ds(c * CHUNK, CHUNK)]

        def wait_g(slot, rows, sem):
            pltpu.make_async_copy(table_hbm.at[idx_v.at[slot]], rows, sem).wait()

        def wait_w(rows, sem):
            pltpu.make_async_copy(rows, out_at(0), sem).wait()

        def run(first, nchunks):
            pltpu.sync_copy(idx_hbm.at[first], idx_v.at[0])
            pltpu.async_copy(table_hbm.at[idx_v.at[0]], rows0, g0)

            def body(i, _):
                a = first + 2 * i

                @pl.when(i > 0)
                def _():
                    wait_w(rows1, w1)
                pltpu.sync_copy(idx_hbm.at[a + 1], idx_v.at[1])
                pltpu.async_copy(table_hbm.at[idx_v.at[1]], rows1, g1)

                wait_g(0, rows0, g0)
                pltpu.async_copy(rows0, out_at(a), w0)

                wait_w(rows0, w0)
                pltpu.sync_copy(idx_hbm.at[a + 2], idx_v.at[0])
                pltpu.async_copy(table_hbm.at[idx_v.at[0]], rows0, g0)

                wait_g(1, rows1, g1)
                pltpu.async_copy(rows1, out_at(a + 1), w1)
                return _

            lax.fori_loop(0, (nchunks - 2) // 2, body, None)
            last = first + nchunks - 1
            wait_w(rows1, w1)
            pltpu.sync_copy(idx_hbm.at[last], idx_v.at[1])
            pltpu.async_copy(table_hbm.at[idx_v.at[1]], rows1, g1)
            wait_g(0, rows0, g0)
            pltpu.async_copy(rows0, out_at(last - 1), w0)
            wait_g(1, rows1, g1)
            pltpu.async_copy(rows1, out_at(last), w1)
            wait_w(rows0, w0)
            wait_w(rows1, w1)

        @pl.when(cid == 0)
        def _():
            run(sid * CH_C0, CH_C0)

        @pl.when(cid == 1)
        def _():
            run(16 * CH_C0 + sid * CH_C1, CH_C1)

    return gather


def _conv_body(u_ref, g_ref, wbt_ref, bb_ref, dun_ref, ban_ref, dvn_ref,
               u2_ref, v2_ref, C, CO):
    u = u_ref[...]
    g = g_ref[...]
    urep = jnp.broadcast_to(u[:, None, :], (NODE_TILE, K, C))
    urep = urep.reshape(NODE_TILE * K, C)
    m = jnp.maximum(urep + g, 0.0)
    y = jnp.maximum(_dot(m, wbt_ref[...]) + bb_ref[...], 0.0)
    xn = jnp.max(y.reshape(NODE_TILE, K, C), axis=1)
    u2_ref[...] = _dot(xn, dun_ref[...]) + ban_ref[...]
    v2_ref[...] = _dot(xn, dvn_ref[...])


def _run_conv1(u1, g1rows, We1b, bb1, We2a, ba2):
    A1 = We2a[:, :128]
    A2 = We2a[:, 128:]
    body = functools.partial(_conv_body, C=128, CO=256)
    return pl.pallas_call(
        body,
        grid=(N_PAD // NODE_TILE,),
        in_specs=[
            pl.BlockSpec((NODE_TILE, 128), lambda i: (i, 0)),
            pl.BlockSpec((NODE_TILE * K, 128), lambda i: (i, 0)),
            pl.BlockSpec((128, 128), lambda i: (0, 0)),
            pl.BlockSpec((1, 128), lambda i: (0, 0)),
            pl.BlockSpec((128, 256), lambda i: (0, 0)),
            pl.BlockSpec((1, 256), lambda i: (0, 0)),
            pl.BlockSpec((128, 256), lambda i: (0, 0)),
        ],
        out_specs=[
            pl.BlockSpec((NODE_TILE, 256), lambda i: (i, 0)),
            pl.BlockSpec((NODE_TILE, 256), lambda i: (i, 0)),
        ],
        out_shape=[
            jax.ShapeDtypeStruct((N_PAD, 256), jnp.float32),
            jax.ShapeDtypeStruct((N_PAD, 256), jnp.float32),
        ],
    )(u1, g1rows, We1b.T, bb1[None], (A1 - A2).T, ba2[None], A2.T)


def _conv2_body(u_ref, g_ref, wbt_ref, bb_ref, x2_ref):
    C = 256
    u = u_ref[...]
    g = g_ref[...]
    urep = jnp.broadcast_to(u[:, None, :], (NODE_TILE, K, C))
    urep = urep.reshape(NODE_TILE * K, C)
    m = jnp.maximum(urep + g, 0.0)
    y = jnp.maximum(_dot(m, wbt_ref[...]) + bb_ref[...], 0.0)
    x2_ref[...] = jnp.max(y.reshape(NODE_TILE, K, C), axis=1)


def _run_conv2(u2, g2rows, We2b, bb2):
    return pl.pallas_call(
        _conv2_body,
        grid=(N_PAD // NODE_TILE,),
        in_specs=[
            pl.BlockSpec((NODE_TILE, 256), lambda i: (i, 0)),
            pl.BlockSpec((NODE_TILE * K, 256), lambda i: (i, 0)),
            pl.BlockSpec((256, 256), lambda i: (0, 0)),
            pl.BlockSpec((1, 256), lambda i: (0, 0)),
        ],
        out_specs=pl.BlockSpec((NODE_TILE, 256), lambda i: (i, 0)),
        out_shape=jax.ShapeDtypeStruct((N_PAD, 256), jnp.float32),
    )(u2, g2rows, We2b.T, bb2[None])


def _head_body(x_ref, wf1t_ref, bf1_ref, wf2t_ref, bf2_ref, mu_ref, lv_ref):
    x = x_ref[...]
    mean = jnp.mean(x, axis=1)
    mx = jnp.max(x, axis=1)
    xg = jnp.concatenate([mean, mx], axis=1)
    z = jnp.maximum(_dot(xg, wf1t_ref[...]) + bf1_ref[...], 0.0)
    lat = _dot(z, wf2t_ref[...]) + bf2_ref[...]
    mu_ref[...] = lat[:, :256]
    lv_ref[...] = lat[:, 256:]


def _run_head(x2, Wf1, bf1, Wf2, bf2):
    return pl.pallas_call(
        _head_body,
        out_shape=[
            jax.ShapeDtypeStruct((B, 256), jnp.float32),
            jax.ShapeDtypeStruct((B, 256), jnp.float32),
        ],
    )(x2, Wf1.T, bf1[None], Wf2.T, bf2[None])


def kernel(x, W1, b1, g1, be1, W2, b2, g2, be2,
           We1a, ba1, We1b, bb1, We2a, ba2, We2b, bb2,
           Wf1, bf1, Wf2, bf2):
    x2d = x.reshape(BN, 10)
    h, u1, v1 = _run_mlp(x2d, W1, b1, g1, be1, W2, b2, g2, be2, We1a, ba1)

    pos = h[:, :3].reshape(B, N, 3).transpose(0, 2, 1)
    posT = jnp.pad(pos, ((0, 0), (0, 1), (0, NCOL - N)))

    u1b = jnp.pad(u1.reshape(B, N, 128), ((0, 0), (0, N_PAD - N), (0, 0)))
    v1b = jnp.pad(v1.reshape(B, N, 128), ((0, 0), (0, N_PAD - N), (0, 0)))

    gather128 = _make_gather(128)
    gather256 = _make_gather(256)

    x2s = []
    for b in range(B):
        idx_b = _run_knn(posT[b:b + 1])
        srcl = idx_b[0].transpose(1, 0).reshape(E_PAD)
        srcl = srcl.reshape(NCHUNKS, CHUNK)
        g1rows = gather128(srcl, v1b[b])
        u2, v2 = _run_conv1(u1b[b], g1rows, We1b, bb1, We2a, ba2)
        g2rows = gather256(srcl, v2)
        x2 = _run_conv2(u2, g2rows, We2b, bb2)
        x2s.append(x2[:N])
    x2all = jnp.stack(x2s)
    mu_, lv = _run_head(x2all, Wf1, bf1, Wf2, bf2)
    return (mu_, lv)


# speedup vs baseline: 1.3796x; 1.0989x over previous
import functools

import jax
import jax.numpy as jnp
from jax import lax
from jax.experimental import pallas as pl
from jax.experimental.pallas import tpu as pltpu
from jax.experimental.pallas import tpu_sc as plsc

B, N, K = 4, 2500, 16
EPS = 1e-5
BN = B * N
NODE_TILE = 256
N_PAD = 2560
E = BN * K
E_PAD = N_PAD * K
NCOL = 2560
ROW_TILE = 512

NUM_WORKERS = 32
CHUNK = 128
NCHUNKS = E_PAD // CHUNK
CH_C0 = 10
CH_C1 = (NCHUNKS // 16) - CH_C0


def _dot(a, b):
    return jnp.dot(a, b, preferred_element_type=jnp.float32)


def _mlp_body(x_ref, w1t_ref, b1_ref, g1_ref, be1_ref,
              w2t_ref, b2_ref, g2_ref, be2_ref,
              du1_ref, dv1_ref, ba1_ref,
              h_ref, u1_ref, v1_ref):
    x = x_ref[...]
    h1 = _dot(x, w1t_ref[...]) + b1_ref[...]
    mu = jnp.mean(h1, axis=0, keepdims=True)
    var = jnp.mean(h1 * h1, axis=0, keepdims=True) - mu * mu
    a1 = jnp.maximum(g1_ref[...] * (h1 - mu) * jax.lax.rsqrt(var + EPS)
                     + be1_ref[...], 0.0)
    h2 = _dot(a1, w2t_ref[...]) + b2_ref[...]
    mu2 = jnp.mean(h2, axis=0, keepdims=True)
    var2 = jnp.mean(h2 * h2, axis=0, keepdims=True) - mu2 * mu2
    h = jnp.maximum(g2_ref[...] * (h2 - mu2) * jax.lax.rsqrt(var2 + EPS)
                    + be2_ref[...], 0.0)
    h_ref[...] = h
    u1_ref[...] = _dot(h, du1_ref[...]) + ba1_ref[...]
    v1_ref[...] = _dot(h, dv1_ref[...])


def _run_mlp(x2d, W1, b1, g1, be1, W2, b2, g2, be2, We1a, ba1):
    A1 = We1a[:, :128]
    A2 = We1a[:, 128:]
    du1 = (A1 - A2).T
    dv1 = A2.T
    xp = jnp.pad(x2d, ((0, 0), (0, 6)))
    return pl.pallas_call(
        _mlp_body,
        out_shape=[
            jax.ShapeDtypeStruct((BN, 128), jnp.float32),
            jax.ShapeDtypeStruct((BN, 128), jnp.float32),
            jax.ShapeDtypeStruct((BN, 128), jnp.float32),
        ],
    )(xp, jnp.pad(W1.T, ((0, 6), (0, 0))), b1[None], g1[None], be1[None],
      W2.T, b2[None], g2[None], be2[None], du1, dv1, ba1[None])


def _knn_body(pos_r_ref, pos_c_ref, out_ref):
    b = pl.program_id(0)
    rt = pl.program_id(1)
    pr = pos_r_ref[0]
    pc = pos_c_ref[0]
    sq_r = jnp.sum(pr * pr, axis=0)[:, None]
    sq_c = jnp.sum(pc * pc, axis=0)[None, :]
    d2 = sq_r + sq_c - 2.0 * lax.dot_general(
        pr, pc, (((0,), (0,)), ((), ())),
        preferred_element_type=jnp.float32)
    col = lax.broadcasted_iota(jnp.int32, (ROW_TILE, NCOL), 1)
    row = lax.broadcasted_iota(jnp.int32, (ROW_TILE, NCOL), 0) + rt * ROW_TILE
    bad = (col >= N) | (col == row)
    s = jnp.where(bad, 1e30, d2)
    base = b * N
    for k in range(K):
        rmin = jnp.min(s, axis=1, keepdims=True)
        cand = jnp.where(s == rmin, col, jnp.int32(2 ** 30))
        idx = jnp.min(cand, axis=1, keepdims=True)
        out_ref[0, k, :] = idx[:, 0] + base
        s = jnp.where(col == idx, 1e30, s)


def _run_knn(posT):
    grid = (1, NCOL // ROW_TILE)
    return pl.pallas_call(
        _knn_body,
        grid=grid,
        in_specs=[
            pl.BlockSpec((1, 4, ROW_TILE), lambda b, rt: (b, 0, rt)),
            pl.BlockSpec((1, 4, NCOL), lambda b, rt: (b, 0, 0)),
        ],
        out_specs=pl.BlockSpec((1, K, ROW_TILE), lambda b, rt: (b, 0, rt)),
        out_shape=jax.ShapeDtypeStruct((1, K, NCOL), jnp.int32),
    )(posT, posT)


def _make_gather(C):
    info = plsc.get_sparse_core_info()
    nc, ns = info.num_cores, info.num_subcores
    mesh = plsc.VectorSubcoreMesh(core_axis_name="c", subcore_axis_name="s")

    @functools.partial(
        pl.kernel,
        out_type=jax.ShapeDtypeStruct((E_PAD, C), jnp.float32),
        mesh=mesh,
        scratch_types=[
            pltpu.VMEM((2, CHUNK), jnp.int32),
            pltpu.VMEM((CHUNK, C), jnp.float32),
            pltpu.VMEM((CHUNK, C), jnp.float32),
            pltpu.SemaphoreType.DMA,
            pltpu.SemaphoreType.DMA,
            pltpu.SemaphoreType.DMA,
            pltpu.SemaphoreType.DMA,
        ],
    )
    def gather(idx_hbm, table_hbm, out_hbm, idx_v, rows0, rows1, g0, g1, w0, w1):
        sid = lax.axis_index("s")
        cid = lax.axis_index("c")

        def out_at(c):
            return out_hbm.at[pl.ds(c * CHUNK, CHUNK)]

        def wait_g(slot, rows, sem):
            pltpu.make_async_copy(table_hbm.at[idx_v.at[slot]], rows, sem).wait()

        def wait_w(rows, sem):
            pltpu.make_async_copy(rows, out_at(0), sem).wait()

        def run(first, nchunks):
            pltpu.sync_copy(idx_hbm.at[first], idx_v.at[0])
            pltpu.async_copy(table_hbm.at[idx_v.at[0]], rows0, g0)

            def body(i, _):
                a = first + 2 * i

                @pl.when(i > 0)
                def _():
                    wait_w(rows1, w1)
                pltpu.sync_copy(idx_hbm.at[a + 1], idx_v.at[1])
                pltpu.async_copy(table_hbm.at[idx_v.at[1]], rows1, g1)

                wait_g(0, rows0, g0)
                pltpu.async_copy(rows0, out_at(a), w0)

                wait_w(rows0, w0)
                pltpu.sync_copy(idx_hbm.at[a + 2], idx_v.at[0])
                pltpu.async_copy(table_hbm.at[idx_v.at[0]], rows0, g0)

                wait_g(1, rows1, g1)
                pltpu.async_copy(rows1, out_at(a + 1), w1)
                return _

            lax.fori_loop(0, (nchunks - 2) // 2, body, None)
            last = first + nchunks - 1
            wait_w(rows1, w1)
            pltpu.sync_copy(idx_hbm.at[last], idx_v.at[1])
            pltpu.async_copy(table_hbm.at[idx_v.at[1]], rows1, g1)
            wait_g(0, rows0, g0)
            pltpu.async_copy(rows0, out_at(last - 1), w0)
            wait_g(1, rows1, g1)
            pltpu.async_copy(rows1, out_at(last), w1)
            wait_w(rows0, w0)
            wait_w(rows1, w1)

        @pl.when(cid == 0)
        def _():
            run(sid * CH_C0, CH_C0)

        @pl.when(cid == 1)
        def _():
            run(16 * CH_C0 + sid * CH_C1, CH_C1)

    return gather


def _conv_body(u_ref, g_ref, wbt_ref, bb_ref, dun_ref, ban_ref, dvn_ref,
               u2_ref, v2_ref, C, CO):
    u = u_ref[...]
    g = g_ref[...]
    urep = jnp.broadcast_to(u[:, None, :], (NODE_TILE, K, C))
    urep = urep.reshape(NODE_TILE * K, C)
    m = jnp.maximum(urep + g, 0.0)
    y = _dot(m.astype(jnp.bfloat16), wbt_ref[...].astype(jnp.bfloat16))
    y = jnp.maximum(y + bb_ref[...], 0.0)
    xn = jnp.max(y.reshape(NODE_TILE, K, C), axis=1)
    u2_ref[...] = _dot(xn, dun_ref[...]) + ban_ref[...]
    v2_ref[...] = _dot(xn, dvn_ref[...])


def _run_conv1(u1, g1rows, We1b, bb1, We2a, ba2):
    A1 = We2a[:, :128]
    A2 = We2a[:, 128:]
    body = functools.partial(_conv_body, C=128, CO=256)
    return pl.pallas_call(
        body,
        grid=(N_PAD // NODE_TILE,),
        in_specs=[
            pl.BlockSpec((NODE_TILE, 128), lambda i: (i, 0)),
            pl.BlockSpec((NODE_TILE * K, 128), lambda i: (i, 0)),
            pl.BlockSpec((128, 128), lambda i: (0, 0)),
            pl.BlockSpec((1, 128), lambda i: (0, 0)),
            pl.BlockSpec((128, 256), lambda i: (0, 0)),
            pl.BlockSpec((1, 256), lambda i: (0, 0)),
            pl.BlockSpec((128, 256), lambda i: (0, 0)),
        ],
        out_specs=[
            pl.BlockSpec((NODE_TILE, 256), lambda i: (i, 0)),
            pl.BlockSpec((NODE_TILE, 256), lambda i: (i, 0)),
        ],
        out_shape=[
            jax.ShapeDtypeStruct((N_PAD, 256), jnp.float32),
            jax.ShapeDtypeStruct((N_PAD, 256), jnp.float32),
        ],
    )(u1, g1rows, We1b.T, bb1[None], (A1 - A2).T, ba2[None], A2.T)


def _conv2_body(u_ref, g_ref, wbt_ref, bb_ref, x2_ref):
    C = 256
    u = u_ref[...]
    g = g_ref[...]
    urep = jnp.broadcast_to(u[:, None, :], (NODE_TILE, K, C))
    urep = urep.reshape(NODE_TILE * K, C)
    m = jnp.maximum(urep + g, 0.0)
    y = _dot(m.astype(jnp.bfloat16), wbt_ref[...].astype(jnp.bfloat16))
    y = jnp.maximum(y + bb_ref[...], 0.0)
    x2_ref[...] = jnp.max(y.reshape(NODE_TILE, K, C), axis=1)


def _run_conv2(u2, g2rows, We2b, bb2):
    return pl.pallas_call(
        _conv2_body,
        grid=(N_PAD // NODE_TILE,),
        in_specs=[
            pl.BlockSpec((NODE_TILE, 256), lambda i: (i, 0)),
            pl.BlockSpec((NODE_TILE * K, 256), lambda i: (i, 0)),
            pl.BlockSpec((256, 256), lambda i: (0, 0)),
            pl.BlockSpec((1, 256), lambda i: (0, 0)),
        ],
        out_specs=pl.BlockSpec((NODE_TILE, 256), lambda i: (i, 0)),
        out_shape=jax.ShapeDtypeStruct((N_PAD, 256), jnp.float32),
    )(u2, g2rows, We2b.T, bb2[None])


def _head_body(x_ref, wf1t_ref, bf1_ref, wf2t_ref, bf2_ref, mu_ref, lv_ref):
    x = x_ref[...]
    mean = jnp.mean(x, axis=1)
    mx = jnp.max(x, axis=1)
    xg = jnp.concatenate([mean, mx], axis=1)
    z = jnp.maximum(_dot(xg, wf1t_ref[...]) + bf1_ref[...], 0.0)
    lat = _dot(z, wf2t_ref[...]) + bf2_ref[...]
    mu_ref[...] = lat[:, :256]
    lv_ref[...] = lat[:, 256:]


def _run_head(x2, Wf1, bf1, Wf2, bf2):
    return pl.pallas_call(
        _head_body,
        out_shape=[
            jax.ShapeDtypeStruct((B, 256), jnp.float32),
            jax.ShapeDtypeStruct((B, 256), jnp.float32),
        ],
    )(x2, Wf1.T, bf1[None], Wf2.T, bf2[None])


def kernel(x, W1, b1, g1, be1, W2, b2, g2, be2,
           We1a, ba1, We1b, bb1, We2a, ba2, We2b, bb2,
           Wf1, bf1, Wf2, bf2):
    x2d = x.reshape(BN, 10)
    h, u1, v1 = _run_mlp(x2d, W1, b1, g1, be1, W2, b2, g2, be2, We1a, ba1)

    pos = h[:, :3].reshape(B, N, 3).transpose(0, 2, 1)
    posT = jnp.pad(pos, ((0, 0), (0, 1), (0, NCOL - N)))

    u1b = jnp.pad(u1.reshape(B, N, 128), ((0, 0), (0, N_PAD - N), (0, 0)))
    v1b = jnp.pad(v1.reshape(B, N, 128), ((0, 0), (0, N_PAD - N), (0, 0)))

    gather128 = _make_gather(128)
    gather256 = _make_gather(256)

    x2s = []
    for b in range(B):
        idx_b = _run_knn(posT[b:b + 1])
        srcl = idx_b[0].transpose(1, 0).reshape(E_PAD)
        srcl = srcl.reshape(NCHUNKS, CHUNK)
        g1rows = gather128(srcl, v1b[b])
        u2, v2 = _run_conv1(u1b[b], g1rows, We1b, bb1, We2a, ba2)
        g2rows = gather256(srcl, v2)
        x2 = _run_conv2(u2, g2rows, We2b, bb2)
        x2s.append(x2[:N])
    x2all = jnp.stack(x2s)
    mu_, lv = _run_head(x2all, Wf1, bf1, Wf2, bf2)
    return (mu_, lv)
